# Initial kernel scaffold; baseline (speedup 1.0000x reference)
import jax
import jax.numpy as jnp
from jax.experimental import pallas as pl


def kernel(x, edge_index, edge_attr, layer_tab, color_tab, relsize_tab, W1, b1, g1, be1, W2, b2, g2, be2, W3, b3, g3, be3, Wc, bc):
    raise NotImplementedError("write your pallas kernel here")


# speedup vs baseline: 6.0799x; 6.0799x over previous
import functools

import jax
import jax.numpy as jnp
from jax import lax
from jax.experimental import pallas as pl
from jax.experimental.pallas import tpu as pltpu
from jax.experimental.pallas import tpu_sc as plsc

F32 = jnp.float32
I32 = jnp.int32

N = 10000
E = 160000
EP = 163840
CB = 128
RW = EP // CB // 32
NPAD = 10240
SLC = 640
BN = 400
NBLK = N // BN


def _mesh():
    return plsc.VectorSubcoreMesh(
        core_axis_name="c", subcore_axis_name="s", num_cores=2, num_subcores=16)


def _splat16(val):
    return jnp.full((16,), val, dtype=I32)


_GDN = lax.GatherDimensionNumbers(
    offset_dims=(), collapsed_slice_dims=(0,), start_index_map=(0,))


def _dyn_splat(vec16, lane):
    idx = jnp.full((16, 1), lane, I32)
    return lax.gather(vec16, idx, _GDN, (1,),
                      mode=lax.GatherScatterMode.PROMISE_IN_BOUNDS)


def _sc_deg(dst2, w2, z0):
    def body(dst_hbm, w_hbm, z_hbm, out_hbm, dst_v, w_v, acc):
        c = lax.axis_index("c")
        s = lax.axis_index("s")
        base = (c * 16 + s) * RW

        def zb(k, carry):
            pltpu.sync_copy(z_hbm.at[0], acc.at[pl.ds(s * SLC + k * CB, CB)])
            return carry
        lax.fori_loop(0, SLC // CB, zb, 0)
        pltpu.sync_copy(dst_hbm.at[pl.ds(base, RW)], dst_v)
        pltpu.sync_copy(w_hbm.at[pl.ds(base, RW)], w_v)
        plsc.subcore_barrier()

        def jb(j, carry):
            pltpu.sync_copy(w_v.at[j], acc.at[dst_v.at[j]], add=True)
            return carry
        lax.fori_loop(0, RW, jb, 0)
        plsc.subcore_barrier()
        pltpu.sync_copy(acc.at[pl.ds(s * SLC, SLC)],
                        out_hbm.at[c, pl.ds(s * SLC, SLC)])

    return pl.kernel(
        body,
        out_type=jax.ShapeDtypeStruct((2, NPAD), F32),
        mesh=_mesh(),
        scratch_types=[
            pltpu.VMEM((RW, CB), I32),
            pltpu.VMEM((RW, CB), F32),
            pltpu.VMEM_SHARED((NPAD,), F32),
        ],
    )(dst2, w2, z0)


def _sc_ws(src2, dst2, w2, dinv):
    def body(src_hbm, dst_hbm, w_hbm, dinv_hbm, out_hbm,
             src_v, dst_v, w_v, a_v, b_v, o_v, sem):
        c = lax.axis_index("c")
        s = lax.axis_index("s")
        base = (c * 16 + s) * RW
        pltpu.sync_copy(src_hbm.at[pl.ds(base, RW)], src_v)
        pltpu.sync_copy(dst_hbm.at[pl.ds(base, RW)], dst_v)
        pltpu.sync_copy(w_hbm.at[pl.ds(base, RW)], w_v)

        def jb(j, carry):
            pltpu.async_copy(dinv_hbm.at[src_v.at[j]], a_v, sem).wait()
            pltpu.async_copy(dinv_hbm.at[dst_v.at[j]], b_v, sem).wait()
            for u in range(CB // 16):
                sl = pl.ds(u * 16, 16)
                o_v[sl] = a_v[sl] * b_v[sl] * w_v[j, sl]
            pltpu.sync_copy(o_v, out_hbm.at[base + j])
            return carry
        lax.fori_loop(0, RW, jb, 0)

    return pl.kernel(
        body,
        out_type=jax.ShapeDtypeStruct((EP // CB, CB), F32),
        mesh=_mesh(),
        scratch_types=[
            pltpu.VMEM((RW, CB), I32),
            pltpu.VMEM((RW, CB), I32),
            pltpu.VMEM((RW, CB), F32),
            pltpu.VMEM((CB,), F32),
            pltpu.VMEM((CB,), F32),
            pltpu.VMEM((CB,), F32),
            pltpu.SemaphoreType.DMA,
        ],
    )(src2, dst2, w2, dinv)


def _sc_agg(src2, dst2, ws2, g_list, z0, fb):
    nb = len(g_list)

    def body(*refs):
        (src_hbm, dst_hbm, ws_hbm), rest = refs[:3], refs[3:]
        g_hbm = rest[:nb]
        z_hbm = rest[nb]
        out_hbm = rest[nb + 1:nb + 1 + nb]
        src_v, dst_v, ws_v, rows_v, acc, sem = rest[nb + 1 + nb:]

        c = lax.axis_index("c")
        s = lax.axis_index("s")
        base = (c * 16 + s) * RW
        pltpu.sync_copy(src_hbm.at[pl.ds(base, RW)], src_v)
        pltpu.sync_copy(dst_hbm.at[pl.ds(base, RW)], dst_v)
        pltpu.sync_copy(ws_hbm.at[pl.ds(base * CB, RW * CB)], ws_v)

        for f in range(nb):
            pltpu.sync_copy(z_hbm, acc.at[pl.ds(s * SLC, SLC)])
            plsc.subcore_barrier()

            def jb(j, carry):
                pltpu.async_copy(g_hbm[f].at[src_v.at[j]], rows_v, sem).wait()

                def rb(r, rc):
                    base_r = j * CB + (r // 16) * 16
                    wvec = ws_v[pl.ds(base_r, 16)]
                    wspl = _dyn_splat(wvec, r % 16)
                    for u in range(fb // 16):
                        sl = pl.ds(u * 16, 16)
                        rows_v[r, sl] = rows_v[r, sl] * wspl
                    return rc
                lax.fori_loop(0, CB, rb, 0)
                pltpu.sync_copy(rows_v, acc.at[dst_v.at[j]], add=True)
                return carry
            lax.fori_loop(0, RW, jb, 0)
            plsc.subcore_barrier()

            @pl.when(s < 15)
            def _():
                pltpu.sync_copy(acc.at[pl.ds(s * SLC, SLC)],
                                out_hbm[f].at[c, pl.ds(s * SLC, SLC)])

            @pl.when(s == 15)
            def _():
                pltpu.sync_copy(acc.at[pl.ds(15 * SLC, N - 15 * SLC)],
                                out_hbm[f].at[c, pl.ds(15 * SLC, N - 15 * SLC)])

    out = pl.kernel(
        body,
        out_type=[jax.ShapeDtypeStruct((2, N, fb), F32)] * nb,
        mesh=_mesh(),
        scratch_types=[
            pltpu.VMEM((RW, CB), I32),
            pltpu.VMEM((RW, CB), I32),
            pltpu.VMEM((RW * CB,), F32),
            pltpu.VMEM((CB, fb), F32),
            pltpu.VMEM_SHARED((NPAD, fb), F32),
            pltpu.SemaphoreType.DMA,
        ],
    )(src2, dst2, ws2.reshape(EP), *g_list, z0)
    return list(out) if isinstance(out, (list, tuple)) else [out]


def _tc_smm(a, b):
    m, k = a.shape
    _, p = b.shape

    def body(a_ref, b_ref, o_ref):
        o_ref[...] = jnp.dot(a_ref[...], b_ref[...],
                             preferred_element_type=F32)

    return pl.pallas_call(
        body,
        out_shape=jax.ShapeDtypeStruct((m, p), F32),
    )(a, b)


def _tc_dinv(degp):
    def body(d_ref, dinv_ref, inv_ref):
        deg = d_ref[0:1, :] + d_ref[1:2, :] + 1.0
        dinv_ref[...] = lax.rsqrt(deg)
        inv_ref[...] = 1.0 / deg

    return pl.pallas_call(
        body,
        out_shape=[jax.ShapeDtypeStruct((1, NPAD), F32),
                   jax.ShapeDtypeStruct((1, NPAD), F32)],
    )(degp)


def _tc_mm1(x, w_res, t_c, t_16):
    def body(x_ref, wr_ref, tc_ref, t16_ref, o0, o1, o2, o3):
        xb = x_ref[...]
        acc = jnp.dot(xb[:, 1:1001], wr_ref[...], preferred_element_type=F32)
        iota256 = lax.broadcasted_iota(I32, (BN, 256), 1)
        for k in range(3):
            ci = xb[:, 1002 + k:1003 + k].astype(I32)
            oh = (iota256 == ci).astype(F32)
            acc += jnp.dot(oh, tc_ref[:, 512 * k:512 * (k + 1)],
                           preferred_element_type=F32)
        li = xb[:, 0:1].astype(I32)
        rsi = jnp.round(xb[:, 1001:1002] * 10.0).astype(I32)
        iota16 = lax.broadcasted_iota(I32, (BN, 16), 1)
        oh16 = ((iota16 == li) | (iota16 == rsi + 3)).astype(F32)
        acc += jnp.dot(oh16, t16_ref[...], preferred_element_type=F32)
        o0[...] = acc[:, 0:128]
        o1[...] = acc[:, 128:256]
        o2[...] = acc[:, 256:384]
        o3[...] = acc[:, 384:512]

    outs = pl.pallas_call(
        body,
        grid=(NBLK,),
        in_specs=[
            pl.BlockSpec((BN, 1005), lambda i: (i, 0)),
            pl.BlockSpec((1000, 512), lambda i: (0, 0)),
            pl.BlockSpec((256, 1536), lambda i: (0, 0)),
            pl.BlockSpec((16, 512), lambda i: (0, 0)),
        ],
        out_specs=[pl.BlockSpec((BN, 128), lambda i: (i, 0))] * 4,
        out_shape=[jax.ShapeDtypeStruct((N, 128), F32)] * 4,
    )(x, w_res, t_c, t_16)
    return list(outs)


def _bn_coeffs(ssum_ref, ssqc_ref, gam_ref, bet_ref):
    mean = ssum_ref[...] * (1.0 / N)
    var = ssqc_ref[...] * (1.0 / N)
    a = gam_ref[...] * lax.rsqrt(var + 1e-5)
    b = bet_ref[...] - mean * a
    return a, b


def _act_block(s_refs, h_refs, inv, a, b, fb):
    pieces = []
    for f in range(len(s_refs)):
        sb = s_refs[f][...]
        u = sb[0] + sb[1] + h_refs[f][...] * inv
        z = u * a[0:1, f * fb:(f + 1) * fb] + b[0:1, f * fb:(f + 1) * fb]
        pieces.append(jnp.where(z > 0, z, 0.01 * z))
    return pieces[0] if len(pieces) == 1 else jnp.concatenate(pieces, axis=1)


def _tc_stats(s_list, h_list, inv_col, fb):
    nb = len(s_list)
    d = nb * fb

    def body(*refs):
        s_refs = refs[:nb]
        h_refs = refs[nb:2 * nb]
        inv_ref = refs[2 * nb]
        ssum_ref, ssqc_ref = refs[2 * nb + 1:]
        i = pl.program_id(0)

        @pl.when(i == 0)
        def _():
            ssum_ref[...] = jnp.zeros_like(ssum_ref)
            ssqc_ref[...] = jnp.zeros_like(ssqc_ref)

        inv = inv_ref[...]
        for f in range(nb):
            sb = s_refs[f][...]
            u = sb[0] + sb[1] + h_refs[f][...] * inv
            sl = pl.ds(f * fb, fb)

            @pl.when(i < NBLK)
            def _():
                ssum_ref[0:1, sl] += jnp.sum(u, axis=0, keepdims=True)

            @pl.when(i >= NBLK)
            def _():
                uc = u - ssum_ref[0:1, sl] * (1.0 / N)
                ssqc_ref[0:1, sl] += jnp.sum(uc * uc, axis=0, keepdims=True)

    return pl.pallas_call(
        body,
        grid=(2 * NBLK,),
        in_specs=(
            [pl.BlockSpec((2, BN, fb), lambda i: (0, i % NBLK, 0))] * nb
            + [pl.BlockSpec((BN, fb), lambda i: (i % NBLK, 0))] * nb
            + [pl.BlockSpec((BN, 1), lambda i: (i % NBLK, 0))]
        ),
        out_specs=[pl.BlockSpec((1, d), lambda i: (0, 0))] * 2,
        out_shape=[jax.ShapeDtypeStruct((1, d), F32)] * 2,
    )(*s_list, *h_list, inv_col)


def _tc_mm_next(s_list, h_list, inv_col, ssum, ssq, gam, bet, w, fb_out):
    nb = len(s_list)
    fb = s_list[0].shape[-1]
    d_in = nb * fb
    d_out = w.shape[1]
    nb_out = d_out // fb_out

    def body(*refs):
        s_refs = refs[:nb]
        h_refs = refs[nb:2 * nb]
        inv_ref, ssum_ref, ssq_ref, gam_ref, bet_ref, w_ref = \
            refs[2 * nb:2 * nb + 6]
        o_refs = refs[2 * nb + 6:]
        a, b = _bn_coeffs(ssum_ref, ssq_ref, gam_ref, bet_ref)
        act = _act_block(s_refs, h_refs, inv_ref[...], a, b, fb)
        res = jnp.dot(act, w_ref[...], preferred_element_type=F32)
        for o in range(nb_out):
            o_refs[o][...] = res[:, o * fb_out:(o + 1) * fb_out]

    outs = pl.pallas_call(
        body,
        grid=(NBLK,),
        in_specs=(
            [pl.BlockSpec((2, BN, fb), lambda i: (0, i, 0))] * nb
            + [pl.BlockSpec((BN, fb), lambda i: (i, 0))] * nb
            + [pl.BlockSpec((BN, 1), lambda i: (i, 0)),
               pl.BlockSpec((1, d_in), lambda i: (0, 0)),
               pl.BlockSpec((1, d_in), lambda i: (0, 0)),
               pl.BlockSpec((1, d_in), lambda i: (0, 0)),
               pl.BlockSpec((1, d_in), lambda i: (0, 0)),
               pl.BlockSpec((d_in, d_out), lambda i: (0, 0))]
        ),
        out_specs=[pl.BlockSpec((BN, fb_out), lambda i: (i, 0))] * nb_out,
        out_shape=[jax.ShapeDtypeStruct((N, fb_out), F32)] * nb_out,
    )(*s_list, *h_list, inv_col, ssum, ssq, gam, bet, w)
    return list(outs)


def _tc_final(s3, h3, inv_col, ssum, ssq, gam, bet, wc, bc, fb):
    def body(s_ref, h_ref, inv_ref, ssum_ref, ssq_ref, gam_ref, bet_ref,
             wc_ref, bc_ref, o_ref):
        a, b = _bn_coeffs(ssum_ref, ssq_ref, gam_ref, bet_ref)
        act = _act_block([s_ref], [h_ref], inv_ref[...], a, b, fb)
        o_ref[...] = jnp.dot(act, wc_ref[...],
                             preferred_element_type=F32) + bc_ref[...]

    return pl.pallas_call(
        body,
        grid=(NBLK,),
        in_specs=[
            pl.BlockSpec((2, BN, fb), lambda i: (0, i, 0)),
            pl.BlockSpec((BN, fb), lambda i: (i, 0)),
            pl.BlockSpec((BN, 1), lambda i: (i, 0)),
            pl.BlockSpec((1, fb), lambda i: (0, 0)),
            pl.BlockSpec((1, fb), lambda i: (0, 0)),
            pl.BlockSpec((1, fb), lambda i: (0, 0)),
            pl.BlockSpec((1, fb), lambda i: (0, 0)),
            pl.BlockSpec((fb, 3), lambda i: (0, 0)),
            pl.BlockSpec((1, 3), lambda i: (0, 0)),
        ],
        out_specs=pl.BlockSpec((BN, 3), lambda i: (i, 0)),
        out_shape=jax.ShapeDtypeStruct((N, 3), F32),
    )(s3, h3, inv_col, ssum, ssq, gam, bet, wc, bc)


def kernel(x, edge_index, edge_attr, layer_tab, color_tab, relsize_tab,
           W1, b1, g1, be1, W2, b2, g2, be2, W3, b3, g3, be3, Wc, bc):
    del b1, b2, b3

    src = edge_index[0]
    dst = edge_index[1]
    npad = EP - E
    pidx = (jnp.arange(npad, dtype=I32) % N).astype(I32)
    src2 = jnp.concatenate([src, pidx]).reshape(EP // CB, CB)
    dst2 = jnp.concatenate([dst, pidx]).reshape(EP // CB, CB)
    w2 = jnp.concatenate([edge_attr, jnp.zeros((npad,), F32)]
                         ).reshape(EP // CB, CB)
    z128 = jnp.zeros((SLC, 128), F32)

    degp = _sc_deg(dst2, w2, z128)
    dinv_r, inv_r = _tc_dinv(degp)
    dinv = dinv_r.reshape(NPAD)[:N]
    inv_col = inv_r.reshape(NPAD)[:N].reshape(N, 1)
    ws2 = _sc_ws(src2, dst2, w2, dinv)

    t_le = _tc_smm(jnp.pad(layer_tab, ((0, 5), (0, 0))), W1[0:250])[:3]
    t_rs = _tc_smm(jnp.pad(relsize_tab, ((0, 5), (0, 0))), W1[1250:1500])[:11]
    w1c = jnp.concatenate([W1[1500 + 85 * k:1585 + 85 * k] for k in range(3)],
                          axis=1)
    t_c = _tc_smm(color_tab, w1c)
    t_16 = jnp.concatenate([t_le, t_rs, jnp.zeros((2, 512), F32)], axis=0)

    h1 = _tc_mm1(x, W1[250:1250], t_c, t_16)
    s1 = _sc_agg(src2, dst2, ws2, h1, z128, 128)
    ss1, sq1 = _tc_stats(s1, h1, inv_col, 128)

    h2 = _tc_mm_next(s1, h1, inv_col, ss1, sq1,
                     g1.reshape(1, 512), be1.reshape(1, 512), W2, 128)
    s2 = _sc_agg(src2, dst2, ws2, h2, z128, 128)
    ss2, sq2 = _tc_stats(s2, h2, inv_col, 128)

    w3p = jnp.pad(W3, ((0, 0), (0, 64)))
    h3 = _tc_mm_next(s2, h2, inv_col, ss2, sq2,
                     g2.reshape(1, 256), be2.reshape(1, 256), w3p, 128)
    s3 = _sc_agg(src2, dst2, ws2, h3, z128, 128)
    ss3, sq3 = _tc_stats(s3, h3, inv_col, 128)

    return _tc_final(s3[0], h3[0], inv_col, ss3, sq3,
                     jnp.pad(g3, (0, 64)).reshape(1, 128),
                     jnp.pad(be3, (0, 64)).reshape(1, 128),
                     jnp.pad(Wc, ((0, 64), (0, 0))),
                     bc.reshape(1, 3), 128)


# speedup vs baseline: 8.1987x; 1.3485x over previous
import functools

import jax
import jax.numpy as jnp
from jax import lax
from jax.experimental import pallas as pl
from jax.experimental.pallas import tpu as pltpu
from jax.experimental.pallas import tpu_sc as plsc

F32 = jnp.float32
I32 = jnp.int32

N = 10000
E = 160000
EP = 163840
CB = 128
RW = EP // CB // 32
NPAD = 10240
SLC = 640
BN = 400
NBLK = N // BN


def _mesh():
    return plsc.VectorSubcoreMesh(
        core_axis_name="c", subcore_axis_name="s", num_cores=2, num_subcores=16)


def _splat16(val):
    return jnp.full((16,), val, dtype=I32)


_GDN = lax.GatherDimensionNumbers(
    offset_dims=(), collapsed_slice_dims=(0,), start_index_map=(0,))


def _dyn_splat(vec16, lane):
    idx = jnp.full((16, 1), lane, I32)
    return lax.gather(vec16, idx, _GDN, (1,),
                      mode=lax.GatherScatterMode.PROMISE_IN_BOUNDS)


def _sc_deg(dst2, w2, z0):
    def body(dst_hbm, w_hbm, z_hbm, out_hbm, dst_v, w_v, acc):
        c = lax.axis_index("c")
        s = lax.axis_index("s")
        base = (c * 16 + s) * RW

        def zb(k, carry):
            pltpu.sync_copy(z_hbm.at[0], acc.at[pl.ds(s * SLC + k * CB, CB)])
            return carry
        lax.fori_loop(0, SLC // CB, zb, 0)
        pltpu.sync_copy(dst_hbm.at[pl.ds(base, RW)], dst_v)
        pltpu.sync_copy(w_hbm.at[pl.ds(base, RW)], w_v)
        plsc.subcore_barrier()

        def jb(j, carry):
            pltpu.sync_copy(w_v.at[j], acc.at[dst_v.at[j]], add=True)
            return carry
        lax.fori_loop(0, RW, jb, 0)
        plsc.subcore_barrier()
        pltpu.sync_copy(acc.at[pl.ds(s * SLC, SLC)],
                        out_hbm.at[c, pl.ds(s * SLC, SLC)])

    return pl.kernel(
        body,
        out_type=jax.ShapeDtypeStruct((2, NPAD), F32),
        mesh=_mesh(),
        scratch_types=[
            pltpu.VMEM((RW, CB), I32),
            pltpu.VMEM((RW, CB), F32),
            pltpu.VMEM_SHARED((NPAD,), F32),
        ],
    )(dst2, w2, z0)


def _sc_ws(src2, dst2, w2, dinv):
    def body(src_hbm, dst_hbm, w_hbm, dinv_hbm, out_hbm,
             src_v, dst_v, w_v, a_v, b_v, o_v, sem):
        c = lax.axis_index("c")
        s = lax.axis_index("s")
        base = (c * 16 + s) * RW
        pltpu.sync_copy(src_hbm.at[pl.ds(base, RW)], src_v)
        pltpu.sync_copy(dst_hbm.at[pl.ds(base, RW)], dst_v)
        pltpu.sync_copy(w_hbm.at[pl.ds(base, RW)], w_v)

        def jb(j, carry):
            pltpu.async_copy(dinv_hbm.at[src_v.at[j]], a_v, sem).wait()
            pltpu.async_copy(dinv_hbm.at[dst_v.at[j]], b_v, sem).wait()
            for u in range(CB // 16):
                sl = pl.ds(u * 16, 16)
                o_v[sl] = a_v[sl] * b_v[sl] * w_v[j, sl]
            pltpu.sync_copy(o_v, out_hbm.at[base + j])
            return carry
        lax.fori_loop(0, RW, jb, 0)

    return pl.kernel(
        body,
        out_type=jax.ShapeDtypeStruct((EP // CB, CB), F32),
        mesh=_mesh(),
        scratch_types=[
            pltpu.VMEM((RW, CB), I32),
            pltpu.VMEM((RW, CB), I32),
            pltpu.VMEM((RW, CB), F32),
            pltpu.VMEM((CB,), F32),
            pltpu.VMEM((CB,), F32),
            pltpu.VMEM((CB,), F32),
            pltpu.SemaphoreType.DMA,
        ],
    )(src2, dst2, w2, dinv)


def _sc_agg(src2, dst2, ws2, g_list, z0, fb):
    nb = len(g_list)

    nbuf = 2

    def body(*refs):
        (src_hbm, dst_hbm, ws_hbm), rest = refs[:3], refs[3:]
        g_hbm = rest[:nb]
        z_hbm = rest[nb]
        out_hbm = rest[nb + 1:nb + 1 + nb]
        rest = rest[nb + 1 + nb:]
        src_v, dst_v, ws_v = rest[:3]
        bufs = rest[3:3 + nbuf]
        acc = rest[3 + nbuf]
        gsems = rest[4 + nbuf:4 + 2 * nbuf]
        ssems = rest[4 + 2 * nbuf:4 + 3 * nbuf]

        c = lax.axis_index("c")
        s = lax.axis_index("s")
        base = (c * 16 + s) * RW
        pltpu.sync_copy(src_hbm.at[pl.ds(base, RW)], src_v)
        pltpu.sync_copy(dst_hbm.at[pl.ds(base, RW)], dst_v)
        pltpu.sync_copy(ws_hbm.at[pl.ds(base * CB, RW * CB)], ws_v)

        def scale(buf, cidx):
            wbase = cidx * CB

            @plsc.parallel_loop(0, CB, unroll=4)
            def _(r):
                wvec = ws_v[pl.ds(wbase + (r // 16) * 16, 16)]
                wspl = _dyn_splat(wvec, r % 16)
                for u in range(fb // 16):
                    sl = pl.ds(u * 16, 16)
                    buf[r, sl] = buf[r, sl] * wspl

        for f in range(nb):
            pltpu.sync_copy(z_hbm, acc.at[pl.ds(s * SLC, SLC)])
            plsc.subcore_barrier()

            def tb(t, carry):
                c0 = t * nbuf
                gds = [pltpu.async_copy(g_hbm[f].at[src_v.at[c0 + k]],
                                        bufs[k], gsems[k])
                       for k in range(nbuf)]
                sds = []
                for k in range(nbuf):
                    gds[k].wait()
                    scale(bufs[k], c0 + k)
                    sds.append(pltpu.async_copy(
                        bufs[k], acc.at[dst_v.at[c0 + k]], ssems[k],
                        add=True))
                for d in sds:
                    d.wait()
                return carry
            lax.fori_loop(0, RW // nbuf, tb, 0)
            plsc.subcore_barrier()

            @pl.when(s < 15)
            def _():
                pltpu.sync_copy(acc.at[pl.ds(s * SLC, SLC)],
                                out_hbm[f].at[c, pl.ds(s * SLC, SLC)])

            @pl.when(s == 15)
            def _():
                pltpu.sync_copy(acc.at[pl.ds(15 * SLC, N - 15 * SLC)],
                                out_hbm[f].at[c, pl.ds(15 * SLC, N - 15 * SLC)])

    out = pl.kernel(
        body,
        out_type=[jax.ShapeDtypeStruct((2, N, fb), F32)] * nb,
        mesh=_mesh(),
        scratch_types=[
            pltpu.VMEM((RW, CB), I32),
            pltpu.VMEM((RW, CB), I32),
            pltpu.VMEM((RW * CB,), F32),
        ] + [pltpu.VMEM((CB, fb), F32)] * nbuf + [
            pltpu.VMEM_SHARED((NPAD, fb), F32),
        ] + [pltpu.SemaphoreType.DMA] * (2 * nbuf),
    )(src2, dst2, ws2.reshape(EP), *g_list, z0)
    return list(out) if isinstance(out, (list, tuple)) else [out]


def _tc_smm(a, b):
    m, k = a.shape
    _, p = b.shape

    def body(a_ref, b_ref, o_ref):
        o_ref[...] = jnp.dot(a_ref[...], b_ref[...],
                             preferred_element_type=F32)

    return pl.pallas_call(
        body,
        out_shape=jax.ShapeDtypeStruct((m, p), F32),
    )(a, b)


def _tc_dinv(degp):
    def body(d_ref, dinv_ref, inv_ref):
        deg = d_ref[0:1, :] + d_ref[1:2, :] + 1.0
        dinv_ref[...] = lax.rsqrt(deg)
        inv_ref[...] = 1.0 / deg

    return pl.pallas_call(
        body,
        out_shape=[jax.ShapeDtypeStruct((1, NPAD), F32),
                   jax.ShapeDtypeStruct((1, NPAD), F32)],
    )(degp)


def _tc_mm1(x, w_res, t_c, t_16):
    def body(x_ref, wr_ref, tc_ref, t16_ref, o0, o1, o2, o3):
        xb = x_ref[...]
        acc = jnp.dot(xb[:, 1:1001], wr_ref[...], preferred_element_type=F32)
        iota256 = lax.broadcasted_iota(I32, (BN, 256), 1)
        for k in range(3):
            ci = xb[:, 1002 + k:1003 + k].astype(I32)
            oh = (iota256 == ci).astype(F32)
            acc += jnp.dot(oh, tc_ref[:, 512 * k:512 * (k + 1)],
                           preferred_element_type=F32)
        li = xb[:, 0:1].astype(I32)
        rsi = jnp.round(xb[:, 1001:1002] * 10.0).astype(I32)
        iota16 = lax.broadcasted_iota(I32, (BN, 16), 1)
        oh16 = ((iota16 == li) | (iota16 == rsi + 3)).astype(F32)
        acc += jnp.dot(oh16, t16_ref[...], preferred_element_type=F32)
        o0[...] = acc[:, 0:128]
        o1[...] = acc[:, 128:256]
        o2[...] = acc[:, 256:384]
        o3[...] = acc[:, 384:512]

    outs = pl.pallas_call(
        body,
        grid=(NBLK,),
        in_specs=[
            pl.BlockSpec((BN, 1005), lambda i: (i, 0)),
            pl.BlockSpec((1000, 512), lambda i: (0, 0)),
            pl.BlockSpec((256, 1536), lambda i: (0, 0)),
            pl.BlockSpec((16, 512), lambda i: (0, 0)),
        ],
        out_specs=[pl.BlockSpec((BN, 128), lambda i: (i, 0))] * 4,
        out_shape=[jax.ShapeDtypeStruct((N, 128), F32)] * 4,
    )(x, w_res, t_c, t_16)
    return list(outs)


def _bn_coeffs(ssum_ref, ssqc_ref, gam_ref, bet_ref):
    mean = ssum_ref[...] * (1.0 / N)
    var = ssqc_ref[...] * (1.0 / N)
    a = gam_ref[...] * lax.rsqrt(var + 1e-5)
    b = bet_ref[...] - mean * a
    return a, b


def _act_block(s_refs, h_refs, inv, a, b, fb):
    pieces = []
    for f in range(len(s_refs)):
        sb = s_refs[f][...]
        u = sb[0] + sb[1] + h_refs[f][...] * inv
        z = u * a[0:1, f * fb:(f + 1) * fb] + b[0:1, f * fb:(f + 1) * fb]
        pieces.append(jnp.where(z > 0, z, 0.01 * z))
    return pieces[0] if len(pieces) == 1 else jnp.concatenate(pieces, axis=1)


def _tc_stats(s_list, h_list, inv_col, fb):
    nb = len(s_list)
    d = nb * fb

    def body(*refs):
        s_refs = refs[:nb]
        h_refs = refs[nb:2 * nb]
        inv_ref = refs[2 * nb]
        ssum_ref, ssqc_ref = refs[2 * nb + 1:]
        i = pl.program_id(0)

        @pl.when(i == 0)
        def _():
            ssum_ref[...] = jnp.zeros_like(ssum_ref)
            ssqc_ref[...] = jnp.zeros_like(ssqc_ref)

        inv = inv_ref[...]
        for f in range(nb):
            sb = s_refs[f][...]
            u = sb[0] + sb[1] + h_refs[f][...] * inv
            sl = pl.ds(f * fb, fb)

            @pl.when(i < NBLK)
            def _():
                ssum_ref[0:1, sl] += jnp.sum(u, axis=0, keepdims=True)

            @pl.when(i >= NBLK)
            def _():
                uc = u - ssum_ref[0:1, sl] * (1.0 / N)
                ssqc_ref[0:1, sl] += jnp.sum(uc * uc, axis=0, keepdims=True)

    return pl.pallas_call(
        body,
        grid=(2 * NBLK,),
        in_specs=(
            [pl.BlockSpec((2, BN, fb), lambda i: (0, i % NBLK, 0))] * nb
            + [pl.BlockSpec((BN, fb), lambda i: (i % NBLK, 0))] * nb
            + [pl.BlockSpec((BN, 1), lambda i: (i % NBLK, 0))]
        ),
        out_specs=[pl.BlockSpec((1, d), lambda i: (0, 0))] * 2,
        out_shape=[jax.ShapeDtypeStruct((1, d), F32)] * 2,
    )(*s_list, *h_list, inv_col)


def _tc_mm_next(s_list, h_list, inv_col, ssum, ssq, gam, bet, w, fb_out):
    nb = len(s_list)
    fb = s_list[0].shape[-1]
    d_in = nb * fb
    d_out = w.shape[1]
    nb_out = d_out // fb_out

    def body(*refs):
        s_refs = refs[:nb]
        h_refs = refs[nb:2 * nb]
        inv_ref, ssum_ref, ssq_ref, gam_ref, bet_ref, w_ref = \
            refs[2 * nb:2 * nb + 6]
        o_refs = refs[2 * nb + 6:]
        a, b = _bn_coeffs(ssum_ref, ssq_ref, gam_ref, bet_ref)
        act = _act_block(s_refs, h_refs, inv_ref[...], a, b, fb)
        res = jnp.dot(act, w_ref[...], preferred_element_type=F32)
        for o in range(nb_out):
            o_refs[o][...] = res[:, o * fb_out:(o + 1) * fb_out]

    outs = pl.pallas_call(
        body,
        grid=(NBLK,),
        in_specs=(
            [pl.BlockSpec((2, BN, fb), lambda i: (0, i, 0))] * nb
            + [pl.BlockSpec((BN, fb), lambda i: (i, 0))] * nb
            + [pl.BlockSpec((BN, 1), lambda i: (i, 0)),
               pl.BlockSpec((1, d_in), lambda i: (0, 0)),
               pl.BlockSpec((1, d_in), lambda i: (0, 0)),
               pl.BlockSpec((1, d_in), lambda i: (0, 0)),
               pl.BlockSpec((1, d_in), lambda i: (0, 0)),
               pl.BlockSpec((d_in, d_out), lambda i: (0, 0))]
        ),
        out_specs=[pl.BlockSpec((BN, fb_out), lambda i: (i, 0))] * nb_out,
        out_shape=[jax.ShapeDtypeStruct((N, fb_out), F32)] * nb_out,
    )(*s_list, *h_list, inv_col, ssum, ssq, gam, bet, w)
    return list(outs)


def _tc_final(s3, h3, inv_col, ssum, ssq, gam, bet, wc, bc, fb):
    def body(s_ref, h_ref, inv_ref, ssum_ref, ssq_ref, gam_ref, bet_ref,
             wc_ref, bc_ref, o_ref):
        a, b = _bn_coeffs(ssum_ref, ssq_ref, gam_ref, bet_ref)
        act = _act_block([s_ref], [h_ref], inv_ref[...], a, b, fb)
        o_ref[...] = jnp.dot(act, wc_ref[...],
                             preferred_element_type=F32) + bc_ref[...]

    return pl.pallas_call(
        body,
        grid=(NBLK,),
        in_specs=[
            pl.BlockSpec((2, BN, fb), lambda i: (0, i, 0)),
            pl.BlockSpec((BN, fb), lambda i: (i, 0)),
            pl.BlockSpec((BN, 1), lambda i: (i, 0)),
            pl.BlockSpec((1, fb), lambda i: (0, 0)),
            pl.BlockSpec((1, fb), lambda i: (0, 0)),
            pl.BlockSpec((1, fb), lambda i: (0, 0)),
            pl.BlockSpec((1, fb), lambda i: (0, 0)),
            pl.BlockSpec((fb, 3), lambda i: (0, 0)),
            pl.BlockSpec((1, 3), lambda i: (0, 0)),
        ],
        out_specs=pl.BlockSpec((BN, 3), lambda i: (i, 0)),
        out_shape=jax.ShapeDtypeStruct((N, 3), F32),
    )(s3, h3, inv_col, ssum, ssq, gam, bet, wc, bc)


def kernel(x, edge_index, edge_attr, layer_tab, color_tab, relsize_tab,
           W1, b1, g1, be1, W2, b2, g2, be2, W3, b3, g3, be3, Wc, bc):
    del b1, b2, b3

    src = edge_index[0]
    dst = edge_index[1]
    npad = EP - E
    pidx = (jnp.arange(npad, dtype=I32) % N).astype(I32)
    src2 = jnp.concatenate([src, pidx]).reshape(EP // CB, CB)
    dst2 = jnp.concatenate([dst, pidx]).reshape(EP // CB, CB)
    w2 = jnp.concatenate([edge_attr, jnp.zeros((npad,), F32)]
                         ).reshape(EP // CB, CB)
    z128 = jnp.zeros((SLC, 128), F32)

    degp = _sc_deg(dst2, w2, z128)
    dinv_r, inv_r = _tc_dinv(degp)
    dinv = dinv_r.reshape(NPAD)[:N]
    inv_col = inv_r.reshape(NPAD)[:N].reshape(N, 1)
    ws2 = _sc_ws(src2, dst2, w2, dinv)

    t_le = _tc_smm(jnp.pad(layer_tab, ((0, 5), (0, 0))), W1[0:250])[:3]
    t_rs = _tc_smm(jnp.pad(relsize_tab, ((0, 5), (0, 0))), W1[1250:1500])[:11]
    w1c = jnp.concatenate([W1[1500 + 85 * k:1585 + 85 * k] for k in range(3)],
                          axis=1)
    t_c = _tc_smm(color_tab, w1c)
    t_16 = jnp.concatenate([t_le, t_rs, jnp.zeros((2, 512), F32)], axis=0)

    h1 = _tc_mm1(x, W1[250:1250], t_c, t_16)
    s1 = _sc_agg(src2, dst2, ws2, h1, z128, 128)
    ss1, sq1 = _tc_stats(s1, h1, inv_col, 128)

    h2 = _tc_mm_next(s1, h1, inv_col, ss1, sq1,
                     g1.reshape(1, 512), be1.reshape(1, 512), W2, 128)
    s2 = _sc_agg(src2, dst2, ws2, h2, z128, 128)
    ss2, sq2 = _tc_stats(s2, h2, inv_col, 128)

    w3p = jnp.pad(W3, ((0, 0), (0, 64)))
    h3 = _tc_mm_next(s2, h2, inv_col, ss2, sq2,
                     g2.reshape(1, 256), be2.reshape(1, 256), w3p, 128)
    s3 = _sc_agg(src2, dst2, ws2, h3, z128, 128)
    ss3, sq3 = _tc_stats(s3, h3, inv_col, 128)

    return _tc_final(s3[0], h3[0], inv_col, ss3, sq3,
                     jnp.pad(g3, (0, 64)).reshape(1, 128),
                     jnp.pad(be3, (0, 64)).reshape(1, 128),
                     jnp.pad(Wc, ((0, 64), (0, 0))),
                     bc.reshape(1, 3), 128)


# speedup vs baseline: 8.2383x; 1.0048x over previous
import functools

import jax
import jax.numpy as jnp
from jax import lax
from jax.experimental import pallas as pl
from jax.experimental.pallas import tpu as pltpu
from jax.experimental.pallas import tpu_sc as plsc

F32 = jnp.float32
I32 = jnp.int32

N = 10000
E = 160000
EP = 163840
CB = 128
RW = EP // CB // 32
NPAD = 10240
SLC = 640
BN = 400
NBLK = N // BN


def _mesh():
    return plsc.VectorSubcoreMesh(
        core_axis_name="c", subcore_axis_name="s", num_cores=2, num_subcores=16)


def _splat16(val):
    return jnp.full((16,), val, dtype=I32)


_GDN = lax.GatherDimensionNumbers(
    offset_dims=(), collapsed_slice_dims=(0,), start_index_map=(0,))


def _dyn_splat(vec16, lane):
    idx = jnp.full((16, 1), lane, I32)
    return lax.gather(vec16, idx, _GDN, (1,),
                      mode=lax.GatherScatterMode.PROMISE_IN_BOUNDS)


def _sc_deg(dst2, w2, z0):
    def body(dst_hbm, w_hbm, z_hbm, out_hbm, dst_v, w_v, acc):
        c = lax.axis_index("c")
        s = lax.axis_index("s")
        base = (c * 16 + s) * RW

        def zb(k, carry):
            pltpu.sync_copy(z_hbm.at[0], acc.at[pl.ds(s * SLC + k * CB, CB)])
            return carry
        lax.fori_loop(0, SLC // CB, zb, 0)
        pltpu.sync_copy(dst_hbm.at[pl.ds(base, RW)], dst_v)
        pltpu.sync_copy(w_hbm.at[pl.ds(base, RW)], w_v)
        plsc.subcore_barrier()

        def jb(j, carry):
            pltpu.sync_copy(w_v.at[j], acc.at[dst_v.at[j]], add=True)
            return carry
        lax.fori_loop(0, RW, jb, 0)
        plsc.subcore_barrier()
        pltpu.sync_copy(acc.at[pl.ds(s * SLC, SLC)],
                        out_hbm.at[c, pl.ds(s * SLC, SLC)])

    return pl.kernel(
        body,
        out_type=jax.ShapeDtypeStruct((2, NPAD), F32),
        mesh=_mesh(),
        scratch_types=[
            pltpu.VMEM((RW, CB), I32),
            pltpu.VMEM((RW, CB), F32),
            pltpu.VMEM_SHARED((NPAD,), F32),
        ],
    )(dst2, w2, z0)


def _sc_ws(src2, dst2, w2, dinv):
    def body(src_hbm, dst_hbm, w_hbm, dinv_hbm, out_hbm,
             src_v, dst_v, w_v, a_v, b_v, o_v, sem):
        c = lax.axis_index("c")
        s = lax.axis_index("s")
        base = (c * 16 + s) * RW
        pltpu.sync_copy(src_hbm.at[pl.ds(base, RW)], src_v)
        pltpu.sync_copy(dst_hbm.at[pl.ds(base, RW)], dst_v)
        pltpu.sync_copy(w_hbm.at[pl.ds(base, RW)], w_v)

        def jb(j, carry):
            pltpu.async_copy(dinv_hbm.at[src_v.at[j]], a_v, sem).wait()
            pltpu.async_copy(dinv_hbm.at[dst_v.at[j]], b_v, sem).wait()
            for u in range(CB // 16):
                sl = pl.ds(u * 16, 16)
                o_v[sl] = a_v[sl] * b_v[sl] * w_v[j, sl]
            pltpu.sync_copy(o_v, out_hbm.at[base + j])
            return carry
        lax.fori_loop(0, RW, jb, 0)

    return pl.kernel(
        body,
        out_type=jax.ShapeDtypeStruct((EP // CB, CB), F32),
        mesh=_mesh(),
        scratch_types=[
            pltpu.VMEM((RW, CB), I32),
            pltpu.VMEM((RW, CB), I32),
            pltpu.VMEM((RW, CB), F32),
            pltpu.VMEM((CB,), F32),
            pltpu.VMEM((CB,), F32),
            pltpu.VMEM((CB,), F32),
            pltpu.SemaphoreType.DMA,
        ],
    )(src2, dst2, w2, dinv)


def _sc_agg(src2, dst2, ws2, g_list, z0, fb):
    nb = len(g_list)

    nbuf = 2

    def body(*refs):
        (src_hbm, dst_hbm, ws_hbm), rest = refs[:3], refs[3:]
        g_hbm = rest[:nb]
        z_hbm = rest[nb]
        out_hbm = rest[nb + 1:nb + 1 + nb]
        rest = rest[nb + 1 + nb:]
        src_v, dst_v, ws_v = rest[:3]
        bufs = rest[3:3 + nbuf]
        acc = rest[3 + nbuf]
        gsems = rest[4 + nbuf:4 + 2 * nbuf]
        ssems = rest[4 + 2 * nbuf:4 + 3 * nbuf]

        c = lax.axis_index("c")
        s = lax.axis_index("s")
        base = (c * 16 + s) * RW
        pltpu.sync_copy(src_hbm.at[pl.ds(base, RW)], src_v)
        pltpu.sync_copy(dst_hbm.at[pl.ds(base, RW)], dst_v)
        pltpu.sync_copy(ws_hbm.at[pl.ds(base * CB, RW * CB)], ws_v)

        def scale(buf, cidx):
            wbase = cidx * CB

            @plsc.parallel_loop(0, CB, step=16)
            def _(r):
                wvec = ws_v[pl.ds(wbase + r, 16)]
                for l in range(16):
                    wspl = _dyn_splat(wvec, l)
                    for u in range(fb // 16):
                        sl = pl.ds(u * 16, 16)
                        buf[r + l, sl] = buf[r + l, sl] * wspl

        for f in range(nb):
            pltpu.sync_copy(z_hbm, acc.at[pl.ds(s * SLC, SLC)])
            plsc.subcore_barrier()

            def tb(t, carry):
                c0 = t * nbuf
                gds = [pltpu.async_copy(g_hbm[f].at[src_v.at[c0 + k]],
                                        bufs[k], gsems[k])
                       for k in range(nbuf)]
                sds = []
                for k in range(nbuf):
                    gds[k].wait()
                    scale(bufs[k], c0 + k)
                    sds.append(pltpu.async_copy(
                        bufs[k], acc.at[dst_v.at[c0 + k]], ssems[k],
                        add=True))
                for d in sds:
                    d.wait()
                return carry
            lax.fori_loop(0, RW // nbuf, tb, 0)
            plsc.subcore_barrier()

            @pl.when(s < 15)
            def _():
                pltpu.sync_copy(acc.at[pl.ds(s * SLC, SLC)],
                                out_hbm[f].at[c, pl.ds(s * SLC, SLC)])

            @pl.when(s == 15)
            def _():
                pltpu.sync_copy(acc.at[pl.ds(15 * SLC, N - 15 * SLC)],
                                out_hbm[f].at[c, pl.ds(15 * SLC, N - 15 * SLC)])

    out = pl.kernel(
        body,
        out_type=[jax.ShapeDtypeStruct((2, N, fb), F32)] * nb,
        mesh=_mesh(),
        scratch_types=[
            pltpu.VMEM((RW, CB), I32),
            pltpu.VMEM((RW, CB), I32),
            pltpu.VMEM((RW * CB,), F32),
        ] + [pltpu.VMEM((CB, fb), F32)] * nbuf + [
            pltpu.VMEM_SHARED((NPAD, fb), F32),
        ] + [pltpu.SemaphoreType.DMA] * (2 * nbuf),
    )(src2, dst2, ws2.reshape(EP), *g_list, z0)
    return list(out) if isinstance(out, (list, tuple)) else [out]


def _tc_smm(a, b):
    m, k = a.shape
    _, p = b.shape

    def body(a_ref, b_ref, o_ref):
        o_ref[...] = jnp.dot(a_ref[...], b_ref[...],
                             preferred_element_type=F32)

    return pl.pallas_call(
        body,
        out_shape=jax.ShapeDtypeStruct((m, p), F32),
    )(a, b)


def _tc_dinv(degp):
    def body(d_ref, dinv_ref, inv_ref):
        deg = d_ref[0:1, :] + d_ref[1:2, :] + 1.0
        dinv_ref[...] = lax.rsqrt(deg)
        inv_ref[...] = 1.0 / deg

    return pl.pallas_call(
        body,
        out_shape=[jax.ShapeDtypeStruct((1, NPAD), F32),
                   jax.ShapeDtypeStruct((1, NPAD), F32)],
    )(degp)


def _tc_mm1(x, w_res, t_c, t_16):
    def body(x_ref, wr_ref, tc_ref, t16_ref, o0, o1, o2, o3):
        xb = x_ref[...]
        acc = jnp.dot(xb[:, 1:1001], wr_ref[...], preferred_element_type=F32)
        iota256 = lax.broadcasted_iota(I32, (BN, 256), 1)
        for k in range(3):
            ci = xb[:, 1002 + k:1003 + k].astype(I32)
            oh = (iota256 == ci).astype(F32)
            acc += jnp.dot(oh, tc_ref[:, 512 * k:512 * (k + 1)],
                           preferred_element_type=F32)
        li = xb[:, 0:1].astype(I32)
        rsi = jnp.round(xb[:, 1001:1002] * 10.0).astype(I32)
        iota16 = lax.broadcasted_iota(I32, (BN, 16), 1)
        oh16 = ((iota16 == li) | (iota16 == rsi + 3)).astype(F32)
        acc += jnp.dot(oh16, t16_ref[...], preferred_element_type=F32)
        o0[...] = acc[:, 0:128]
        o1[...] = acc[:, 128:256]
        o2[...] = acc[:, 256:384]
        o3[...] = acc[:, 384:512]

    outs = pl.pallas_call(
        body,
        grid=(NBLK,),
        in_specs=[
            pl.BlockSpec((BN, 1005), lambda i: (i, 0)),
            pl.BlockSpec((1000, 512), lambda i: (0, 0)),
            pl.BlockSpec((256, 1536), lambda i: (0, 0)),
            pl.BlockSpec((16, 512), lambda i: (0, 0)),
        ],
        out_specs=[pl.BlockSpec((BN, 128), lambda i: (i, 0))] * 4,
        out_shape=[jax.ShapeDtypeStruct((N, 128), F32)] * 4,
    )(x, w_res, t_c, t_16)
    return list(outs)


def _bn_coeffs(ssum_ref, ssqc_ref, gam_ref, bet_ref):
    mean = ssum_ref[...] * (1.0 / N)
    var = ssqc_ref[...] * (1.0 / N)
    a = gam_ref[...] * lax.rsqrt(var + 1e-5)
    b = bet_ref[...] - mean * a
    return a, b


def _act_block(s_refs, h_refs, inv, a, b, fb):
    pieces = []
    for f in range(len(s_refs)):
        sb = s_refs[f][...]
        u = sb[0] + sb[1] + h_refs[f][...] * inv
        z = u * a[0:1, f * fb:(f + 1) * fb] + b[0:1, f * fb:(f + 1) * fb]
        pieces.append(jnp.where(z > 0, z, 0.01 * z))
    return pieces[0] if len(pieces) == 1 else jnp.concatenate(pieces, axis=1)


def _tc_stats(s_list, h_list, inv_col, fb):
    nb = len(s_list)
    d = nb * fb

    def body(*refs):
        s_refs = refs[:nb]
        h_refs = refs[nb:2 * nb]
        inv_ref = refs[2 * nb]
        ssum_ref, ssqc_ref = refs[2 * nb + 1:]
        i = pl.program_id(0)

        @pl.when(i == 0)
        def _():
            ssum_ref[...] = jnp.zeros_like(ssum_ref)
            ssqc_ref[...] = jnp.zeros_like(ssqc_ref)

        inv = inv_ref[...]
        for f in range(nb):
            sb = s_refs[f][...]
            u = sb[0] + sb[1] + h_refs[f][...] * inv
            sl = pl.ds(f * fb, fb)

            @pl.when(i < NBLK)
            def _():
                ssum_ref[0:1, sl] += jnp.sum(u, axis=0, keepdims=True)

            @pl.when(i >= NBLK)
            def _():
                uc = u - ssum_ref[0:1, sl] * (1.0 / N)
                ssqc_ref[0:1, sl] += jnp.sum(uc * uc, axis=0, keepdims=True)

    return pl.pallas_call(
        body,
        grid=(2 * NBLK,),
        in_specs=(
            [pl.BlockSpec((2, BN, fb), lambda i: (0, i % NBLK, 0))] * nb
            + [pl.BlockSpec((BN, fb), lambda i: (i % NBLK, 0))] * nb
            + [pl.BlockSpec((BN, 1), lambda i: (i % NBLK, 0))]
        ),
        out_specs=[pl.BlockSpec((1, d), lambda i: (0, 0))] * 2,
        out_shape=[jax.ShapeDtypeStruct((1, d), F32)] * 2,
    )(*s_list, *h_list, inv_col)


def _tc_mm_next(s_list, h_list, inv_col, ssum, ssq, gam, bet, w, fb_out):
    nb = len(s_list)
    fb = s_list[0].shape[-1]
    d_in = nb * fb
    d_out = w.shape[1]
    nb_out = d_out // fb_out

    def body(*refs):
        s_refs = refs[:nb]
        h_refs = refs[nb:2 * nb]
        inv_ref, ssum_ref, ssq_ref, gam_ref, bet_ref, w_ref = \
            refs[2 * nb:2 * nb + 6]
        o_refs = refs[2 * nb + 6:]
        a, b = _bn_coeffs(ssum_ref, ssq_ref, gam_ref, bet_ref)
        act = _act_block(s_refs, h_refs, inv_ref[...], a, b, fb)
        res = jnp.dot(act, w_ref[...], preferred_element_type=F32)
        for o in range(nb_out):
            o_refs[o][...] = res[:, o * fb_out:(o + 1) * fb_out]

    outs = pl.pallas_call(
        body,
        grid=(NBLK,),
        in_specs=(
            [pl.BlockSpec((2, BN, fb), lambda i: (0, i, 0))] * nb
            + [pl.BlockSpec((BN, fb), lambda i: (i, 0))] * nb
            + [pl.BlockSpec((BN, 1), lambda i: (i, 0)),
               pl.BlockSpec((1, d_in), lambda i: (0, 0)),
               pl.BlockSpec((1, d_in), lambda i: (0, 0)),
               pl.BlockSpec((1, d_in), lambda i: (0, 0)),
               pl.BlockSpec((1, d_in), lambda i: (0, 0)),
               pl.BlockSpec((d_in, d_out), lambda i: (0, 0))]
        ),
        out_specs=[pl.BlockSpec((BN, fb_out), lambda i: (i, 0))] * nb_out,
        out_shape=[jax.ShapeDtypeStruct((N, fb_out), F32)] * nb_out,
    )(*s_list, *h_list, inv_col, ssum, ssq, gam, bet, w)
    return list(outs)


def _tc_final(s3, h3, inv_col, ssum, ssq, gam, bet, wc, bc, fb):
    def body(s_ref, h_ref, inv_ref, ssum_ref, ssq_ref, gam_ref, bet_ref,
             wc_ref, bc_ref, o_ref):
        a, b = _bn_coeffs(ssum_ref, ssq_ref, gam_ref, bet_ref)
        act = _act_block([s_ref], [h_ref], inv_ref[...], a, b, fb)
        o_ref[...] = jnp.dot(act, wc_ref[...],
                             preferred_element_type=F32) + bc_ref[...]

    return pl.pallas_call(
        body,
        grid=(NBLK,),
        in_specs=[
            pl.BlockSpec((2, BN, fb), lambda i: (0, i, 0)),
            pl.BlockSpec((BN, fb), lambda i: (i, 0)),
            pl.BlockSpec((BN, 1), lambda i: (i, 0)),
            pl.BlockSpec((1, fb), lambda i: (0, 0)),
            pl.BlockSpec((1, fb), lambda i: (0, 0)),
            pl.BlockSpec((1, fb), lambda i: (0, 0)),
            pl.BlockSpec((1, fb), lambda i: (0, 0)),
            pl.BlockSpec((fb, 3), lambda i: (0, 0)),
            pl.BlockSpec((1, 3), lambda i: (0, 0)),
        ],
        out_specs=pl.BlockSpec((BN, 3), lambda i: (i, 0)),
        out_shape=jax.ShapeDtypeStruct((N, 3), F32),
    )(s3, h3, inv_col, ssum, ssq, gam, bet, wc, bc)


def kernel(x, edge_index, edge_attr, layer_tab, color_tab, relsize_tab,
           W1, b1, g1, be1, W2, b2, g2, be2, W3, b3, g3, be3, Wc, bc):
    del b1, b2, b3

    src = edge_index[0]
    dst = edge_index[1]
    npad = EP - E
    pidx = (jnp.arange(npad, dtype=I32) % N).astype(I32)
    src2 = jnp.concatenate([src, pidx]).reshape(EP // CB, CB)
    dst2 = jnp.concatenate([dst, pidx]).reshape(EP // CB, CB)
    w2 = jnp.concatenate([edge_attr, jnp.zeros((npad,), F32)]
                         ).reshape(EP // CB, CB)
    z128 = jnp.zeros((SLC, 128), F32)

    degp = _sc_deg(dst2, w2, z128)
    dinv_r, inv_r = _tc_dinv(degp)
    dinv = dinv_r.reshape(NPAD)[:N]
    inv_col = inv_r.reshape(NPAD)[:N].reshape(N, 1)
    ws2 = _sc_ws(src2, dst2, w2, dinv)

    t_le = _tc_smm(jnp.pad(layer_tab, ((0, 5), (0, 0))), W1[0:250])[:3]
    t_rs = _tc_smm(jnp.pad(relsize_tab, ((0, 5), (0, 0))), W1[1250:1500])[:11]
    w1c = jnp.concatenate([W1[1500 + 85 * k:1585 + 85 * k] for k in range(3)],
                          axis=1)
    t_c = _tc_smm(color_tab, w1c)
    t_16 = jnp.concatenate([t_le, t_rs, jnp.zeros((2, 512), F32)], axis=0)

    h1 = _tc_mm1(x, W1[250:1250], t_c, t_16)
    s1 = _sc_agg(src2, dst2, ws2, h1, z128, 128)
    ss1, sq1 = _tc_stats(s1, h1, inv_col, 128)

    h2 = _tc_mm_next(s1, h1, inv_col, ss1, sq1,
                     g1.reshape(1, 512), be1.reshape(1, 512), W2, 128)
    s2 = _sc_agg(src2, dst2, ws2, h2, z128, 128)
    ss2, sq2 = _tc_stats(s2, h2, inv_col, 128)

    w3p = jnp.pad(W3, ((0, 0), (0, 64)))
    h3 = _tc_mm_next(s2, h2, inv_col, ss2, sq2,
                     g2.reshape(1, 256), be2.reshape(1, 256), w3p, 128)
    s3 = _sc_agg(src2, dst2, ws2, h3, z128, 128)
    ss3, sq3 = _tc_stats(s3, h3, inv_col, 128)

    return _tc_final(s3[0], h3[0], inv_col, ss3, sq3,
                     jnp.pad(g3, (0, 64)).reshape(1, 128),
                     jnp.pad(be3, (0, 64)).reshape(1, 128),
                     jnp.pad(Wc, ((0, 64), (0, 0))),
                     bc.reshape(1, 3), 128)
